# baseline (speedup 1.0000x reference)
import functools

import jax
import jax.numpy as jnp
from jax import lax
from jax.experimental import pallas as pl
from jax.experimental.pallas import tpu as pltpu
from jax.experimental.pallas import tpu_sc as plsc

N, D, K = 16384, 256, 8192
BN = 256


def _argmin_body(x_ref, cbt_ref, o_ref):
    x_blk = x_ref[...]
    cbt = cbt_ref[...]
    xy = jax.lax.dot_general(
        x_blk, cbt, (((1,), (0,)), ((), ())),
        precision=jax.lax.Precision.DEFAULT,
        preferred_element_type=jnp.float32,
    )
    x2 = jnp.sum(x_blk * x_blk, axis=1, keepdims=True)
    y2 = jnp.sum(cbt * cbt, axis=0, keepdims=True)
    d2 = jnp.maximum(x2 + y2 - 2.0 * xy, 0.0)
    d = jnp.sqrt(d2)
    d_min = jnp.min(d, axis=1, keepdims=True)
    col = jax.lax.broadcasted_iota(jnp.int32, (BN, K), 1)
    idx = jnp.min(jnp.where(d <= d_min, col, K), axis=1)
    o_ref[...] = idx.astype(jnp.int32)


def _argmin_tc(x, cbt):
    return pl.pallas_call(
        _argmin_body,
        grid=(N // BN,),
        in_specs=[
            pl.BlockSpec((BN, D), lambda i: (i, 0)),
            pl.BlockSpec((D, K), lambda i: (0, 0)),
        ],
        out_specs=pl.BlockSpec((BN,), lambda i: (i,)),
        out_shape=jax.ShapeDtypeStruct((N,), jnp.int32),
    )(x, cbt)


_NC, _NS = 2, 16
_NW = _NC * _NS
_B_PER_W = N // _NW
_CH = 128


@functools.cache
def _make_gather_sc():
    @functools.partial(
        pl.kernel,
        out_type=jax.ShapeDtypeStruct((N, D), jnp.float32),
        mesh=plsc.VectorSubcoreMesh(core_axis_name="c", subcore_axis_name="s"),
        scratch_types=[
            pltpu.VMEM((_CH,), jnp.int32),
            pltpu.VMEM((_CH, D), jnp.float32),
            pltpu.SemaphoreType.DMA,
        ],
    )
    def _gather_sc(table_hbm, idx_hbm, out_hbm, idx_v, rows_v, sem):
        wid = lax.axis_index("s") * _NC + lax.axis_index("c")
        base = wid * _B_PER_W
        for c in range(_B_PER_W // _CH):
            off = base + c * _CH
            pltpu.sync_copy(idx_hbm.at[pl.ds(off, _CH)], idx_v)
            pltpu.async_copy(table_hbm.at[idx_v], rows_v, sem).wait()
            pltpu.sync_copy(rows_v, out_hbm.at[pl.ds(off, _CH)])

    return _gather_sc


def kernel(x, codebook):
    cbt = codebook.T
    indices = _argmin_tc(x, cbt)
    quantized = _make_gather_sc()(codebook, indices)
    return (indices, quantized)


# speedup vs baseline: 1.2370x; 1.2370x over previous
import functools

import jax
import jax.numpy as jnp
from jax import lax
from jax.experimental import pallas as pl
from jax.experimental.pallas import tpu as pltpu
from jax.experimental.pallas import tpu_sc as plsc

N, D, K = 16384, 256, 8192
BN = 512


def _argmin_body(x_ref, cbt_ref, o_ref):
    x_blk = x_ref[...]
    cbt = cbt_ref[...]
    xy = jax.lax.dot_general(
        x_blk, cbt, (((1,), (0,)), ((), ())),
        precision=jax.lax.Precision.DEFAULT,
        preferred_element_type=jnp.float32,
    )
    x2 = jnp.sum(x_blk * x_blk, axis=1, keepdims=True)
    y2 = jnp.sum(cbt * cbt, axis=0, keepdims=True)
    d2 = jnp.maximum(x2 + y2 - 2.0 * xy, 0.0)
    d = jnp.sqrt(d2)
    o_ref[...] = jnp.argmin(d, axis=1).astype(jnp.int32)


def _argmin_tc(x, cbt):
    return pl.pallas_call(
        _argmin_body,
        grid=(N // BN,),
        in_specs=[
            pl.BlockSpec((BN, D), lambda i: (i, 0)),
            pl.BlockSpec((D, K), lambda i: (0, 0)),
        ],
        out_specs=pl.BlockSpec((BN,), lambda i: (i,)),
        out_shape=jax.ShapeDtypeStruct((N,), jnp.int32),
        compiler_params=pltpu.CompilerParams(
            dimension_semantics=("parallel",),
        ),
    )(x, cbt)


_NC, _NS = 2, 16
_NW = _NC * _NS
_B_PER_W = N // _NW
_CH = 128


@functools.cache
def _make_gather_sc():
    @functools.partial(
        pl.kernel,
        out_type=jax.ShapeDtypeStruct((N, D), jnp.float32),
        mesh=plsc.VectorSubcoreMesh(core_axis_name="c", subcore_axis_name="s"),
        scratch_types=[
            pltpu.VMEM((_CH,), jnp.int32),
            pltpu.VMEM((_CH, D), jnp.float32),
            pltpu.SemaphoreType.DMA,
        ],
    )
    def _gather_sc(table_hbm, idx_hbm, out_hbm, idx_v, rows_v, sem):
        wid = lax.axis_index("s") * _NC + lax.axis_index("c")
        base = wid * _B_PER_W
        for c in range(_B_PER_W // _CH):
            off = base + c * _CH
            pltpu.sync_copy(idx_hbm.at[pl.ds(off, _CH)], idx_v)
            pltpu.async_copy(table_hbm.at[idx_v], rows_v, sem).wait()
            pltpu.sync_copy(rows_v, out_hbm.at[pl.ds(off, _CH)])

    return _gather_sc


def kernel(x, codebook):
    cbt = codebook.T
    indices = _argmin_tc(x, cbt)
    quantized = _make_gather_sc()(codebook, indices)
    return (indices, quantized)
